# baseline (speedup 1.0000x reference)
import jax
import jax.numpy as jnp
from jax import lax
from jax.experimental import pallas as pl
from jax.experimental.pallas import tpu as pltpu

_IN = 384
_HID = 1536
_FC1 = 3072
_E = 4
_N = 1568
_BT = 224
_NB = _N // _BT
_MGRID = _E

_INTERPRET = False


def _fused_body(t_ref, x_ref, w1a_ref, w1b_ref, b1_ref, w2a_ref, w2b_ref,
                b2_ref, out_ref, xs_v, ys_v, pos_v, offs_v):
    f32 = jnp.float32
    bf16 = jnp.bfloat16
    i32 = jnp.int32
    s = pl.program_id(0)

    @pl.when(s == 0)
    def _routing():
        t = t_ref[...] % _E
        lane128 = lax.broadcasted_iota(i32, (1, 128), 1)
        oh = (t == lane128).astype(bf16)
        r_io = lax.broadcasted_iota(i32, (_N, _N), 0)
        c_io = lax.broadcasted_iota(i32, (_N, _N), 1)
        tril = (r_io > c_io).astype(bf16)
        csum = jnp.dot(tril, oh, preferred_element_type=f32)
        ohf = oh.astype(f32)
        rank = jnp.sum(csum * ohf, axis=1, keepdims=True)

        counts = jnp.sum(ohf, axis=0, keepdims=True)
        su128 = (lax.broadcasted_iota(i32, (128, 128), 0)
                 < lax.broadcasted_iota(i32, (128, 128), 1)).astype(f32)
        offs = jnp.dot(counts, su128, preferred_element_type=f32,
                       precision=lax.Precision.HIGHEST)
        offs_t = jnp.sum(offs * ohf, axis=1, keepdims=True)
        pos = rank + offs_t
        pos_v[...] = pos.astype(i32)

        pos_row = jnp.sum(jnp.where(r_io == c_io, pos, 0.0), axis=0,
                          keepdims=True).astype(i32)
        pt = (r_io == pos_row).astype(bf16)
        xs_v[...] = jnp.dot(pt, x_ref[...].astype(bf16),
                            preferred_element_type=f32).astype(bf16)

        eye16 = (lax.broadcasted_iota(i32, (16, 128), 0)
                 == lax.broadcasted_iota(i32, (16, 128), 1))
        offs_col = jnp.sum(jnp.where(eye16, offs, 0.0), axis=1,
                           keepdims=True)
        offs_v[...] = jnp.concatenate(
            [offs_col, jnp.zeros((16, 7), f32)], axis=1).astype(i32)

    lo = offs_v[s, 0]
    hi = offs_v[s + 1, 0]
    for b in range(_NB):
        blo = b * _BT

        @pl.when((lo < blo + _BT) & (hi > blo))
        def _(blo=blo):
            x = xs_v[pl.ds(blo, _BT), :]
            h = (jnp.dot(x[:, :_IN // 2], w1a_ref[0, 0].astype(bf16),
                         preferred_element_type=f32)
                 + jnp.dot(x[:, _IN // 2:], w1b_ref[0, 0].astype(bf16),
                           preferred_element_type=f32)
                 + b1_ref[0])
            a = h[:, :_HID]
            g = h[:, _HID:]
            h2 = (a * (g / (1.0 + jnp.exp(-g)))).astype(bf16)
            y = (jnp.dot(h2[:, :_HID // 2], w2a_ref[0, 0].astype(bf16),
                         preferred_element_type=f32)
                 + jnp.dot(h2[:, _HID // 2:], w2b_ref[0, 0].astype(bf16),
                           preferred_element_type=f32)
                 + b2_ref[0])
            rows = blo + lax.broadcasted_iota(i32, (_BT, 1), 0)
            mask = (rows >= lo) & (rows < hi)
            ys_v[pl.ds(blo, _BT), :] = jnp.where(
                mask, y, ys_v[pl.ds(blo, _BT), :]).astype(bf16)

    @pl.when(s == _MGRID - 1)
    def _combine():
        c_io = lax.broadcasted_iota(i32, (_N, _N), 1)
        pc = (pos_v[...] == c_io).astype(bf16)
        out_ref[...] = jnp.dot(pc, ys_v[...],
                               preferred_element_type=f32)


def _we(s):
    return s


def _fused(t_col, x2d, W1, b1, W2, b2):
    W1r = W1.reshape(_E, 2, _IN // 2, _FC1)
    W2r = W2.reshape(_E, 2, _HID // 2, _IN)
    return pl.pallas_call(
        _fused_body,
        grid=(_MGRID,),
        in_specs=[
            pl.BlockSpec((_N, 1), lambda s: (0, 0)),
            pl.BlockSpec((_N, _IN), lambda s: (0, 0)),
            pl.BlockSpec((1, 1, _IN // 2, _FC1), lambda s: (_we(s), 0, 0, 0)),
            pl.BlockSpec((1, 1, _IN // 2, _FC1), lambda s: (_we(s), 1, 0, 0)),
            pl.BlockSpec((1, 1, _FC1), lambda s: (_we(s), 0, 0)),
            pl.BlockSpec((1, 1, _HID // 2, _IN), lambda s: (_we(s), 0, 0, 0)),
            pl.BlockSpec((1, 1, _HID // 2, _IN), lambda s: (_we(s), 1, 0, 0)),
            pl.BlockSpec((1, 1, _IN), lambda s: (_we(s), 0, 0)),
        ],
        out_specs=pl.BlockSpec((_N, _IN), lambda s: (0, 0)),
        scratch_shapes=[pltpu.VMEM((_N, _IN), jnp.bfloat16),
                        pltpu.VMEM((_N, _IN), jnp.bfloat16),
                        pltpu.VMEM((_N, 1), jnp.int32),
                        pltpu.VMEM((16, 8), jnp.int32)],
        out_shape=jax.ShapeDtypeStruct((_N, _IN), jnp.float32),
        interpret=_INTERPRET,
    )(t_col, x2d, W1r, W1r, b1, W2r, W2r, b2)


def kernel(x, leat_t, W1, b1, W2, b2):
    x2d = x.reshape(_N, _IN)
    t_col = leat_t.reshape(_N, 1).astype(jnp.int32)
    out2d = _fused(t_col, x2d, W1, b1.reshape(_E, 1, _FC1), W2,
                   b2.reshape(_E, 1, _IN))
    return out2d.reshape(x.shape[:-1] + (_IN,))


# speedup vs baseline: 1.1014x; 1.1014x over previous
import jax
import jax.numpy as jnp
from jax import lax
from jax.experimental import pallas as pl
from jax.experimental.pallas import tpu as pltpu

_IN = 384
_HID = 1536
_FC1 = 3072
_E = 4
_N = 1568
_BT = 224
_NB = _N // _BT
_MGRID = _E

_INTERPRET = False


def _fused_body(t_ref, x_ref, w1_hbm, b1_ref, w2_hbm, b2_ref, out_ref,
                xs_v, ys_v, pos_v, offs_v, w1_v, w2_v, w1b_v, w2b_v,
                sem1, sem2):
    f32 = jnp.float32
    bf16 = jnp.bfloat16
    i32 = jnp.int32
    s = pl.program_id(0)

    @pl.when(s == 0)
    def _start_weight_dmas():
        for j in range(_E):
            pltpu.make_async_copy(w1_hbm.at[j], w1_v.at[j], sem1.at[j]).start()
            pltpu.make_async_copy(w2_hbm.at[j], w2_v.at[j], sem2.at[j]).start()

    @pl.when(s == 0)
    def _routing():
        t = t_ref[...] % _E
        lane128 = lax.broadcasted_iota(i32, (1, 128), 1)
        oh = (t == lane128).astype(bf16)
        r_io = lax.broadcasted_iota(i32, (_N, _N), 0)
        c_io = lax.broadcasted_iota(i32, (_N, _N), 1)
        tril = (r_io > c_io).astype(bf16)
        csum = jnp.dot(tril, oh, preferred_element_type=f32)
        ohf = oh.astype(f32)
        rank = jnp.sum(csum * ohf, axis=1, keepdims=True)

        counts = jnp.sum(ohf, axis=0, keepdims=True)
        su128 = (lax.broadcasted_iota(i32, (128, 128), 0)
                 < lax.broadcasted_iota(i32, (128, 128), 1)).astype(f32)
        offs = jnp.dot(counts, su128, preferred_element_type=f32,
                       precision=lax.Precision.HIGHEST)
        offs_t = jnp.sum(offs * ohf, axis=1, keepdims=True)
        pos = rank + offs_t
        pos_v[...] = pos.astype(i32)

        pos_row = jnp.sum(jnp.where(r_io == c_io, pos, 0.0), axis=0,
                          keepdims=True).astype(i32)
        pt = (r_io == pos_row).astype(bf16)
        xs_v[...] = jnp.dot(pt, x_ref[...].astype(bf16),
                            preferred_element_type=f32).astype(bf16)

        eye16 = (lax.broadcasted_iota(i32, (16, 128), 0)
                 == lax.broadcasted_iota(i32, (16, 128), 1))
        offs_col = jnp.sum(jnp.where(eye16, offs, 0.0), axis=1,
                           keepdims=True)
        offs_v[...] = jnp.concatenate(
            [offs_col, jnp.zeros((16, 7), f32)], axis=1).astype(i32)

    pltpu.make_async_copy(w1_hbm.at[s], w1_v.at[s], sem1.at[s]).wait()
    pltpu.make_async_copy(w2_hbm.at[s], w2_v.at[s], sem2.at[s]).wait()
    w1b_v[...] = w1_v[s].astype(bf16)
    w2b_v[...] = w2_v[s].astype(bf16)

    lo = offs_v[s, 0]
    hi = offs_v[s + 1, 0]
    for b in range(_NB):
        blo = b * _BT

        @pl.when((lo < blo + _BT) & (hi > blo))
        def _(blo=blo):
            x = xs_v[pl.ds(blo, _BT), :]
            h = (jnp.dot(x, w1b_v[...],
                         preferred_element_type=f32) + b1_ref[0])
            a = h[:, :_HID]
            g = h[:, _HID:]
            h2 = (a * (g / (1.0 + jnp.exp(-g)))).astype(bf16)
            y = (jnp.dot(h2, w2b_v[...],
                         preferred_element_type=f32) + b2_ref[0])
            rows = blo + lax.broadcasted_iota(i32, (_BT, 1), 0)
            mask = (rows >= lo) & (rows < hi)
            ys_v[pl.ds(blo, _BT), :] = jnp.where(
                mask, y, ys_v[pl.ds(blo, _BT), :]).astype(bf16)

    @pl.when(s == _MGRID - 1)
    def _combine():
        c_io = lax.broadcasted_iota(i32, (_N, _N), 1)
        pc = (pos_v[...] == c_io).astype(bf16)
        out_ref[...] = jnp.dot(pc, ys_v[...],
                               preferred_element_type=f32)


def _we(s):
    return s


def _fused(t_col, x2d, W1, b1, W2, b2):
    return pl.pallas_call(
        _fused_body,
        grid=(_MGRID,),
        in_specs=[
            pl.BlockSpec((_N, 1), lambda s: (0, 0)),
            pl.BlockSpec((_N, _IN), lambda s: (0, 0)),
            pl.BlockSpec(memory_space=pl.ANY),
            pl.BlockSpec((1, 1, _FC1), lambda s: (_we(s), 0, 0)),
            pl.BlockSpec(memory_space=pl.ANY),
            pl.BlockSpec((1, 1, _IN), lambda s: (_we(s), 0, 0)),
        ],
        out_specs=pl.BlockSpec((_N, _IN), lambda s: (0, 0)),
        scratch_shapes=[pltpu.VMEM((_N, _IN), jnp.bfloat16),
                        pltpu.VMEM((_N, _IN), jnp.bfloat16),
                        pltpu.VMEM((_N, 1), jnp.int32),
                        pltpu.VMEM((16, 8), jnp.int32),
                        pltpu.VMEM((_E, _IN, _FC1), jnp.float32),
                        pltpu.VMEM((_E, _HID, _IN), jnp.float32),
                        pltpu.VMEM((_IN, _FC1), jnp.bfloat16),
                        pltpu.VMEM((_HID, _IN), jnp.bfloat16),
                        pltpu.SemaphoreType.DMA((_E,)),
                        pltpu.SemaphoreType.DMA((_E,))],
        out_shape=jax.ShapeDtypeStruct((_N, _IN), jnp.float32),
        interpret=_INTERPRET,
    )(t_col, x2d, W1, b1, W2, b2)


def kernel(x, leat_t, W1, b1, W2, b2):
    x2d = x.reshape(_N, _IN)
    t_col = leat_t.reshape(_N, 1).astype(jnp.int32)
    out2d = _fused(t_col, x2d, W1, b1.reshape(_E, 1, _FC1), W2,
                   b2.reshape(_E, 1, _IN))
    return out2d.reshape(x.shape[:-1] + (_IN,))
